# baseline (speedup 1.0000x reference)
import functools

import jax
import jax.numpy as jnp
from jax import lax
from jax.experimental import pallas as pl
from jax.experimental.pallas import tpu as pltpu
from jax.experimental.pallas import tpu_sc as plsc

B, N, C, S = 8, 4096, 128, 32
V = S * S * S
H = S / 2.0
NC, NS, L = 2, 16, 16
NW = NC * NS
PTS_W = (B * N) // NW
ROWS_W = PTS_W * 8
CHUNK = 256
NCHUNK = ROWS_W // CHUNK
ZROW = B * V
TBLK = 512
TROWS = B * V + TBLK
TSTEPS = B * (V // TBLK) + 1


def _table_body(x_ref, o_ref):
    i = pl.program_id(0)

    @pl.when(i < TSTEPS - 1)
    def _():
        eye = (lax.broadcasted_iota(jnp.int32, (C, C), 0)
               == lax.broadcasted_iota(jnp.int32, (C, C), 1)
               ).astype(jnp.float32)
        o_ref[...] = lax.dot_general(
            x_ref[0], eye, (((0,), (0,)), ((), ())),
            preferred_element_type=jnp.float32)

    @pl.when(i == TSTEPS - 1)
    def _():
        o_ref[...] = jnp.zeros((TBLK, C), jnp.float32)


def _build_table(cf):
    x = cf.reshape(B, C, V)
    nb = V // TBLK
    return pl.pallas_call(
        _table_body,
        grid=(TSTEPS,),
        in_specs=[pl.BlockSpec(
            (1, C, TBLK),
            lambda i: (jnp.minimum(i, TSTEPS - 2) // nb, 0,
                       jnp.minimum(i, TSTEPS - 2) % nb))],
        out_specs=pl.BlockSpec((TBLK, C), lambda i: (i, 0)),
        out_shape=jax.ShapeDtypeStruct((TROWS, C), jnp.float32),
    )(x)


@functools.partial(
    pl.kernel,
    out_type=jax.ShapeDtypeStruct((B * N * 8, C), jnp.float32),
    mesh=plsc.VectorSubcoreMesh(core_axis_name="c", subcore_axis_name="s"),
    compiler_params=pltpu.CompilerParams(needs_layout_passes=False),
    scratch_types=[
        pltpu.VMEM((PTS_W * 3,), jnp.float32),
        pltpu.VMEM((ROWS_W,), jnp.int32),
        pltpu.VMEM((CHUNK, C), jnp.float32),
        pltpu.VMEM((CHUNK, C), jnp.float32),
        pltpu.SemaphoreType.DMA,
        pltpu.SemaphoreType.DMA,
        pltpu.SemaphoreType.DMA,
        pltpu.SemaphoreType.DMA,
    ],
)
def _sample(pts_hbm, table_hbm, out_hbm, pts_v, idx_v, buf0, buf1,
            gsem0, gsem1, osem0, osem1):
    wid = lax.axis_index("s") * NC + lax.axis_index("c")
    p0 = wid * PTS_W
    base_row = (p0 // N) * V
    pltpu.sync_copy(pts_hbm.at[pl.ds(p0 * 3, PTS_W * 3)], pts_v)

    lanes = lax.iota(jnp.int32, L)

    def compute_idx(i, carry):
        offs = i * (3 * L) + lanes * 3

        def parts(o, mul):
            p = plsc.load_gather(pts_v, [offs + o]) * H + H
            t = p.astype(jnp.int32)
            lo = t - (p < t.astype(jnp.float32)).astype(jnp.int32)
            up = lo + 1
            return (lo * mul, up * mul,
                    (lo >= 0) & (lo < S), (up >= 0) & (up < S))

        xl, xu, vxl, vxu = parts(0, S * S)
        yl, yu, vyl, vyu = parts(1, S)
        zl, zu, vzl, vzu = parts(2, 1)
        pos0 = i * (8 * L) + lanes * 8
        for j in range(8):
            cx, vx = (xu, vxu) if j & 4 else (xl, vxl)
            cy, vy = (yu, vyu) if j & 2 else (yl, vyl)
            cz, vz = (zu, vzu) if j & 1 else (zl, vzl)
            r = jnp.where(vx & vy & vz, base_row + cx + cy + cz, ZROW)
            plsc.store_scatter(idx_v, [pos0 + j], r)
        return carry

    lax.fori_loop(0, PTS_W // L, compute_idx, 0)

    out0 = wid * ROWS_W

    bufs = (buf0, buf1)
    gsems = (gsem0, gsem1)
    osems = (osem0, osem1)
    gcopies = [None, None]
    ocopies = [None, None]
    for k in range(NCHUNK):
        s = k % 2
        if ocopies[s] is not None:
            ocopies[s].wait()
        gcopies[s] = pltpu.async_copy(
            table_hbm.at[idx_v.at[pl.ds(k * CHUNK, CHUNK)]], bufs[s],
            gsems[s])
        if k > 0:
            s1 = (k - 1) % 2
            gcopies[s1].wait()
            ocopies[s1] = pltpu.async_copy(
                bufs[s1], out_hbm.at[pl.ds(out0 + (k - 1) * CHUNK, CHUNK)],
                osems[s1])
    s = (NCHUNK - 1) % 2
    gcopies[s].wait()
    pltpu.sync_copy(bufs[s], out_hbm.at[pl.ds(out0 + (NCHUNK - 1) * CHUNK,
                                              CHUNK)])
    ocopies[1 - s].wait()


def kernel(ptcloud, cubic_features):
    table = _build_table(cubic_features)
    out = _sample(ptcloud.reshape(B * N * 3), table)
    return out.reshape(B, N, 8, C)


# speedup vs baseline: 1.1379x; 1.1379x over previous
import functools

import jax
import jax.numpy as jnp
from jax import lax
from jax.experimental import pallas as pl
from jax.experimental.pallas import tpu as pltpu
from jax.experimental.pallas import tpu_sc as plsc

B, N, C, S = 8, 4096, 128, 32
V = S * S * S
H = S / 2.0
NC, NS, L = 2, 16, 16
NW = NC * NS
PTS_W = (B * N) // NW
ROWS_W = PTS_W * 8
CHUNK = 256
NCHUNK = ROWS_W // CHUNK
ZROW = B * V
TBLK = 2048
TROWS = B * V + TBLK
TSTEPS = B * (V // TBLK) + 1


def _table_body(x_ref, o_ref):
    i = pl.program_id(0)

    @pl.when(i < TSTEPS - 1)
    def _():
        o_ref[...] = x_ref[0].T

    @pl.when(i == TSTEPS - 1)
    def _():
        o_ref[...] = jnp.zeros((TBLK, C), jnp.float32)


def _build_table(cf):
    x = cf.reshape(B, C, V)
    nb = V // TBLK
    return pl.pallas_call(
        _table_body,
        grid=(TSTEPS,),
        in_specs=[pl.BlockSpec(
            (1, C, TBLK),
            lambda i: (jnp.minimum(i, TSTEPS - 2) // nb, 0,
                       jnp.minimum(i, TSTEPS - 2) % nb))],
        out_specs=pl.BlockSpec((TBLK, C), lambda i: (i, 0)),
        out_shape=jax.ShapeDtypeStruct((TROWS, C), jnp.float32),
    )(x)


@functools.partial(
    pl.kernel,
    out_type=jax.ShapeDtypeStruct((B * N * 8, C), jnp.float32),
    mesh=plsc.VectorSubcoreMesh(core_axis_name="c", subcore_axis_name="s"),
    compiler_params=pltpu.CompilerParams(needs_layout_passes=False),
    scratch_types=[
        pltpu.VMEM((PTS_W * 3,), jnp.float32),
        pltpu.VMEM((ROWS_W,), jnp.int32),
        pltpu.VMEM((CHUNK, C), jnp.float32),
        pltpu.VMEM((CHUNK, C), jnp.float32),
        pltpu.SemaphoreType.DMA,
        pltpu.SemaphoreType.DMA,
        pltpu.SemaphoreType.DMA,
        pltpu.SemaphoreType.DMA,
    ],
)
def _sample(pts_hbm, table_hbm, out_hbm, pts_v, idx_v, buf0, buf1,
            gsem0, gsem1, osem0, osem1):
    wid = lax.axis_index("s") * NC + lax.axis_index("c")
    p0 = wid * PTS_W
    base_row = (p0 // N) * V
    pltpu.sync_copy(pts_hbm.at[pl.ds(p0 * 3, PTS_W * 3)], pts_v)

    lanes = lax.iota(jnp.int32, L)

    def compute_idx(i, carry):
        offs = i * (3 * L) + lanes * 3

        def parts(o, mul):
            p = plsc.load_gather(pts_v, [offs + o]) * H + H
            t = p.astype(jnp.int32)
            lo = t - (p < t.astype(jnp.float32)).astype(jnp.int32)
            up = lo + 1
            return (lo * mul, up * mul,
                    (lo >= 0) & (lo < S), (up >= 0) & (up < S))

        xl, xu, vxl, vxu = parts(0, S * S)
        yl, yu, vyl, vyu = parts(1, S)
        zl, zu, vzl, vzu = parts(2, 1)
        pos0 = i * (8 * L) + lanes * 8
        for j in range(8):
            cx, vx = (xu, vxu) if j & 4 else (xl, vxl)
            cy, vy = (yu, vyu) if j & 2 else (yl, vyl)
            cz, vz = (zu, vzu) if j & 1 else (zl, vzl)
            r = jnp.where(vx & vy & vz, base_row + cx + cy + cz, ZROW)
            plsc.store_scatter(idx_v, [pos0 + j], r)
        return carry

    lax.fori_loop(0, PTS_W // L, compute_idx, 0)

    out0 = wid * ROWS_W

    bufs = (buf0, buf1)
    gsems = (gsem0, gsem1)
    osems = (osem0, osem1)
    gcopies = [None, None]
    ocopies = [None, None]
    for k in range(NCHUNK):
        s = k % 2
        if ocopies[s] is not None:
            ocopies[s].wait()
        gcopies[s] = pltpu.async_copy(
            table_hbm.at[idx_v.at[pl.ds(k * CHUNK, CHUNK)]], bufs[s],
            gsems[s])
        if k > 0:
            s1 = (k - 1) % 2
            gcopies[s1].wait()
            ocopies[s1] = pltpu.async_copy(
                bufs[s1], out_hbm.at[pl.ds(out0 + (k - 1) * CHUNK, CHUNK)],
                osems[s1])
    s = (NCHUNK - 1) % 2
    gcopies[s].wait()
    pltpu.sync_copy(bufs[s], out_hbm.at[pl.ds(out0 + (NCHUNK - 1) * CHUNK,
                                              CHUNK)])
    ocopies[1 - s].wait()


def kernel(ptcloud, cubic_features):
    table = _build_table(cubic_features)
    out = _sample(ptcloud.reshape(B * N * 3), table)
    return out.reshape(B, N, 8, C)


# speedup vs baseline: 1.1816x; 1.0384x over previous
import functools

import jax
import jax.numpy as jnp
from jax import lax
from jax.experimental import pallas as pl
from jax.experimental.pallas import tpu as pltpu
from jax.experimental.pallas import tpu_sc as plsc

B, N, C, S = 8, 4096, 128, 32
V = S * S * S
H = S / 2.0
NC, NS, L = 2, 16, 16
NW = NC * NS
PTS_W = (B * N) // NW
ROWS_W = PTS_W * 8
CHUNK = 256
NCHUNK = ROWS_W // CHUNK
ZROW = B * V
TBLK = 8192
TROWS = B * V + TBLK
TSTEPS = B * (V // TBLK) + 1


def _table_body(x_ref, o_ref):
    i = pl.program_id(0)

    @pl.when(i < TSTEPS - 1)
    def _():
        o_ref[...] = x_ref[0].T

    @pl.when(i == TSTEPS - 1)
    def _():
        o_ref[...] = jnp.zeros((TBLK, C), jnp.float32)


def _build_table(cf):
    x = cf.reshape(B, C, V)
    nb = V // TBLK
    return pl.pallas_call(
        _table_body,
        grid=(TSTEPS,),
        in_specs=[pl.BlockSpec(
            (1, C, TBLK),
            lambda i: (jnp.minimum(i, TSTEPS - 2) // nb, 0,
                       jnp.minimum(i, TSTEPS - 2) % nb))],
        out_specs=pl.BlockSpec((TBLK, C), lambda i: (i, 0)),
        out_shape=jax.ShapeDtypeStruct((TROWS, C), jnp.float32),
    )(x)


@functools.partial(
    pl.kernel,
    out_type=jax.ShapeDtypeStruct((B * N * 8, C), jnp.float32),
    mesh=plsc.VectorSubcoreMesh(core_axis_name="c", subcore_axis_name="s"),
    compiler_params=pltpu.CompilerParams(needs_layout_passes=False),
    scratch_types=[
        pltpu.VMEM((PTS_W * 3,), jnp.float32),
        pltpu.VMEM((ROWS_W,), jnp.int32),
        pltpu.VMEM((CHUNK, C), jnp.float32),
        pltpu.VMEM((CHUNK, C), jnp.float32),
        pltpu.SemaphoreType.DMA,
        pltpu.SemaphoreType.DMA,
        pltpu.SemaphoreType.DMA,
        pltpu.SemaphoreType.DMA,
    ],
)
def _sample(pts_hbm, table_hbm, out_hbm, pts_v, idx_v, buf0, buf1,
            gsem0, gsem1, osem0, osem1):
    wid = lax.axis_index("s") * NC + lax.axis_index("c")
    p0 = wid * PTS_W
    base_row = (p0 // N) * V
    pltpu.sync_copy(pts_hbm.at[pl.ds(p0 * 3, PTS_W * 3)], pts_v)

    lanes = lax.iota(jnp.int32, L)

    def compute_idx(i, carry):
        offs = i * (3 * L) + lanes * 3

        def parts(o, mul):
            p = plsc.load_gather(pts_v, [offs + o]) * H + H
            t = p.astype(jnp.int32)
            lo = t - (p < t.astype(jnp.float32)).astype(jnp.int32)
            up = lo + 1
            return (lo * mul, up * mul,
                    (lo >= 0) & (lo < S), (up >= 0) & (up < S))

        xl, xu, vxl, vxu = parts(0, S * S)
        yl, yu, vyl, vyu = parts(1, S)
        zl, zu, vzl, vzu = parts(2, 1)
        pos0 = i * (8 * L) + lanes * 8
        for j in range(8):
            cx, vx = (xu, vxu) if j & 4 else (xl, vxl)
            cy, vy = (yu, vyu) if j & 2 else (yl, vyl)
            cz, vz = (zu, vzu) if j & 1 else (zl, vzl)
            r = jnp.where(vx & vy & vz, base_row + cx + cy + cz, ZROW)
            plsc.store_scatter(idx_v, [pos0 + j], r)
        return carry

    lax.fori_loop(0, PTS_W // L, compute_idx, 0)

    out0 = wid * ROWS_W

    bufs = (buf0, buf1)
    gsems = (gsem0, gsem1)
    osems = (osem0, osem1)
    gcopies = [None, None]
    ocopies = [None, None]
    for k in range(NCHUNK):
        s = k % 2
        if ocopies[s] is not None:
            ocopies[s].wait()
        gcopies[s] = pltpu.async_copy(
            table_hbm.at[idx_v.at[pl.ds(k * CHUNK, CHUNK)]], bufs[s],
            gsems[s])
        if k > 0:
            s1 = (k - 1) % 2
            gcopies[s1].wait()
            ocopies[s1] = pltpu.async_copy(
                bufs[s1], out_hbm.at[pl.ds(out0 + (k - 1) * CHUNK, CHUNK)],
                osems[s1])
    s = (NCHUNK - 1) % 2
    gcopies[s].wait()
    pltpu.sync_copy(bufs[s], out_hbm.at[pl.ds(out0 + (NCHUNK - 1) * CHUNK,
                                              CHUNK)])
    ocopies[1 - s].wait()


def kernel(ptcloud, cubic_features):
    table = _build_table(cubic_features)
    out = _sample(ptcloud.reshape(B * N * 3), table)
    return out.reshape(B, N, 8, C)
